# baseline (speedup 1.0000x reference)
import dataclasses
import functools

import jax
import jax.numpy as jnp
from jax import lax
from jax.experimental import pallas as pl
from jax.experimental.pallas import tpu as pltpu
from jax.experimental.pallas import tpu_sc as plsc

N = 10000
E = 320000
K = 128

NC = 2
NS = 16
NW = NC * NS
EPW = E // NW
C = 40
NCHUNK = EPW // C
DEPTH = 5
LANES = 16
KP = K // 2
RPT = N // NS
ST = 25
NST = RPT // ST


def _sc_compiler_params():
    cp = pltpu.CompilerParams()
    if "needs_layout_passes" in pltpu.CompilerParams.__dataclass_fields__:
        cp = dataclasses.replace(cp, needs_layout_passes=False)
    if "use_tc_tiling_on_sc" in pltpu.CompilerParams.__dataclass_fields__:
        cp = dataclasses.replace(cp, use_tc_tiling_on_sc=False)
    return cp


_UNROLL = 5


def _rne16(b):
    odd = jnp.bitwise_and(lax.shift_right_logical(b, 16), 1)
    return lax.shift_right_logical(b + 0x7FFF + odd, 16)


def _sc_term1(edge_index, a_vals, emb):
    mesh = plsc.VectorSubcoreMesh(core_axis_name="c", subcore_axis_name="s")

    @functools.partial(
        pl.kernel,
        mesh=mesh,
        compiler_params=_sc_compiler_params(),
        out_type=jax.ShapeDtypeStruct((8, 128), jnp.float32),
        scratch_types=[
            pltpu.VMEM((EPW,), jnp.int32),
            pltpu.VMEM((EPW,), jnp.int32),
            pltpu.VMEM((EPW,), jnp.float32),
            [pltpu.VMEM((C, KP), jnp.int32) for _ in range(2 * DEPTH)],
            [pltpu.VMEM((ST, K), jnp.float32) for _ in range(4)],
            [pltpu.VMEM((ST, KP), jnp.int32) for _ in range(2)],
            pltpu.VMEM((LANES,), jnp.float32),
            pltpu.VMEM_SHARED((N, KP), jnp.int32),
            [pltpu.SemaphoreType.DMA for _ in range(2 * DEPTH)],
            [pltpu.SemaphoreType.DMA for _ in range(6)],
        ],
    )
    def k(ei_hbm, a_hbm, emb_hbm, out_hbm,
          sidx, didx, av, rowbufs, stins, stouts, accv, table, sems, stsems):
        sid = lax.axis_index("s")
        wid = sid * NC + lax.axis_index("c")
        base = wid * EPW
        row0 = sid * RPT

        pltpu.sync_copy(ei_hbm.at[0, pl.ds(base, EPW)], sidx)
        pltpu.sync_copy(ei_hbm.at[1, pl.ds(base, EPW)], didx)
        pltpu.sync_copy(a_hbm.at[pl.ds(base, EPW)], av)
        accv[...] = jnp.zeros((LANES,), jnp.float32)

        def st_in(c, ibuf, isem):
            pltpu.async_copy(emb_hbm.at[pl.ds(row0 + c * ST, ST)], ibuf, isem)

        def st_in_wait(c, ibuf, isem):
            pltpu.make_async_copy(
                emb_hbm.at[pl.ds(row0 + c * ST, ST)], ibuf, isem).wait()

        def st_pack(ibuf, obuf):
            def prow(r, carry):
                for j in range(KP // LANES):
                    lo = plsc.bitcast(ibuf[r, pl.ds(j * LANES, LANES)],
                                      jnp.int32)
                    hi = plsc.bitcast(ibuf[r, pl.ds(KP + j * LANES, LANES)],
                                      jnp.int32)
                    obuf[r, pl.ds(j * LANES, LANES)] = jnp.bitwise_or(
                        _rne16(lo), lax.shift_left(_rne16(hi), 16))
                return carry

            lax.fori_loop(0, ST, prow, 0)

        def st_out(c, obuf, osem):
            pltpu.async_copy(obuf, table.at[pl.ds(row0 + c * ST, ST)], osem)

        def st_out_wait(c, obuf, osem):
            pltpu.make_async_copy(
                obuf, table.at[pl.ds(row0 + c * ST, ST)], osem).wait()

        for c in range(4):
            st_in(c, stins[c], stsems[c])
        for c in range(NST):
            st_in_wait(c, stins[c % 4], stsems[c % 4])
            if c >= 2:
                st_out_wait(c - 2, stouts[c % 2], stsems[4 + c % 2])
            st_pack(stins[c % 4], stouts[c % 2])
            if c + 4 < NST:
                st_in(c + 4, stins[c % 4], stsems[c % 4])
            st_out(c, stouts[c % 2], stsems[4 + c % 2])
        st_out_wait(NST - 2, stouts[(NST - 2) % 2], stsems[4 + (NST - 2) % 2])
        st_out_wait(NST - 1, stouts[(NST - 1) % 2], stsems[4 + (NST - 1) % 2])
        plsc.subcore_barrier()

        bufs = tuple((rowbufs[2 * b], rowbufs[2 * b + 1],
                      sems[2 * b], sems[2 * b + 1]) for b in range(DEPTH))

        def issue(ci, sbuf, dbuf, ssem, dsem):
            pltpu.async_copy(table.at[sidx.at[pl.ds(ci * C, C)]], sbuf, ssem)
            pltpu.async_copy(table.at[didx.at[pl.ds(ci * C, C)]], dbuf, dsem)

        def wait(ci, sbuf, dbuf, ssem, dsem):
            pltpu.make_async_copy(
                table.at[sidx.at[pl.ds(ci * C, C)]], sbuf, ssem).wait()
            pltpu.make_async_copy(
                table.at[didx.at[pl.ds(ci * C, C)]], dbuf, dsem).wait()

        def compute(ci, sbuf, dbuf):
            def edge_group(g, acc):
                e0 = g * _UNROLL
                for u in range(_UNROLL):
                    e = e0 + u
                    t = jnp.zeros((LANES,), jnp.float32)
                    for j in range(KP // LANES):
                        sv = plsc.bitcast(
                            sbuf[e, pl.ds(j * LANES, LANES)], jnp.bfloat16)
                        dv = plsc.bitcast(
                            dbuf[e, pl.ds(j * LANES, LANES)], jnp.bfloat16)
                        p0, p1 = plsc.unpack(
                            sv * dv, format=plsc.PackFormat.INTERLEAVED)
                        t = t + p0 + p1
                    ab = plsc.load_gather(
                        av, [jnp.full((LANES,), ci * C + e, jnp.int32)])
                    acc = acc + ab * t
                return acc

            acc = lax.fori_loop(0, C // _UNROLL, edge_group,
                                jnp.zeros((LANES,), jnp.float32))
            accv[...] = accv[...] + acc

        for b in range(DEPTH):
            issue(b, *bufs[b])

        @pl.loop(0, NCHUNK, step=DEPTH)
        def _chunk(ci):
            for b in range(DEPTH):
                sbuf, dbuf, ssem, dsem = bufs[b]
                wait(ci + b, sbuf, dbuf, ssem, dsem)
                compute(ci + b, sbuf, dbuf)

                @pl.when(ci + b + DEPTH < NCHUNK)
                def _():
                    issue(ci + b + DEPTH, sbuf, dbuf, ssem, dsem)

        pltpu.sync_copy(accv,
                        out_hbm.at[wid % 8, pl.ds((wid // 8) * LANES, LANES)])

    return k(edge_index, a_vals, emb)


_BR = 1000


def _gram_body(e_ref, d_ref, ptp_ref, t2_ref):
    @pl.when(pl.program_id(0) == 0)
    def _():
        ptp_ref[...] = jnp.zeros((K, K), jnp.float32)
        t2_ref[0, 0] = 0.0

    blk = e_ref[...]
    ptp_ref[...] += lax.dot_general(blk, blk, (((0,), (0,)), ((), ())),
                                    preferred_element_type=jnp.float32)
    rs = jnp.sum(blk * blk, axis=1)
    drow = d_ref[pl.ds(pl.program_id(0), 1), :]
    t2_ref[0, 0] += jnp.sum(drow[0, :] * rs)


def _tc_gram(emb, d2):
    return pl.pallas_call(
        _gram_body,
        grid=(N // _BR,),
        in_specs=[
            pl.BlockSpec((_BR, K), lambda i: (i, 0)),
            pl.BlockSpec((N // _BR, _BR), lambda i: (0, 0)),
        ],
        out_specs=[
            pl.BlockSpec((K, K), lambda i: (0, 0)),
            pl.BlockSpec(memory_space=pltpu.SMEM),
        ],
        out_shape=[
            jax.ShapeDtypeStruct((K, K), jnp.float32),
            jax.ShapeDtypeStruct((1, 1), jnp.float32),
        ],
    )(emb, d2)


def _combine_body(ptp_ref, part_ref, t2_ref, out_ref):
    ptp = ptp_ref[...]
    term1 = jnp.sum(part_ref[...][:, :NW // 8 * LANES])
    term2 = t2_ref[0, 0]
    n = jnp.sqrt(jnp.sum(ptp * ptp))
    row = lax.broadcasted_iota(jnp.int32, (K, K), 0)
    col = lax.broadcasted_iota(jnp.int32, (K, K), 1)
    eye = jnp.where(row == col, jnp.float32(1.0), jnp.float32(0.0))
    m = ptp / n - eye / jnp.sqrt(jnp.float32(K))
    penalty = jnp.sqrt(jnp.sum(m * m))
    out_ref[0, 0] = -(term1 / term2) + penalty


def _tc_combine(ptp, partials, t2):
    return pl.pallas_call(
        _combine_body,
        in_specs=[
            pl.BlockSpec((K, K), lambda: (0, 0)),
            pl.BlockSpec((8, 128), lambda: (0, 0)),
            pl.BlockSpec(memory_space=pltpu.SMEM),
        ],
        out_specs=pl.BlockSpec(memory_space=pltpu.SMEM),
        out_shape=jax.ShapeDtypeStruct((1, 1), jnp.float32),
    )(ptp, partials, t2)


def kernel(embeddings, edge_index, edge_weight, normalized_A_values, D_values):
    del edge_weight
    partials = _sc_term1(edge_index.astype(jnp.int32), normalized_A_values,
                         embeddings)
    ptp, t2 = _tc_gram(embeddings, D_values.reshape(N // _BR, _BR))
    out = _tc_combine(ptp, partials, t2)
    return out[0, 0]


# speedup vs baseline: 1.0709x; 1.0709x over previous
import dataclasses
import functools

import jax
import jax.numpy as jnp
from jax import lax
from jax.experimental import pallas as pl
from jax.experimental.pallas import tpu as pltpu
from jax.experimental.pallas import tpu_sc as plsc

N = 10000
E = 320000
K = 128

NC = 2
NS = 16
NW = NC * NS
EPW = E // NW
C = 40
NCHUNK = EPW // C
DEPTH = 5
LANES = 16
KP = K // 2
RPT = N // NS
ST = 25
NST = RPT // ST


def _sc_compiler_params():
    cp = pltpu.CompilerParams()
    if "needs_layout_passes" in pltpu.CompilerParams.__dataclass_fields__:
        cp = dataclasses.replace(cp, needs_layout_passes=False)
    if "use_tc_tiling_on_sc" in pltpu.CompilerParams.__dataclass_fields__:
        cp = dataclasses.replace(cp, use_tc_tiling_on_sc=False)
    return cp


_UNROLL = 5


def _rne16(b):
    odd = jnp.bitwise_and(lax.shift_right_logical(b, 16), 1)
    return lax.shift_right_logical(b + 0x7FFF + odd, 16)


def _sc_term1(edge_index, a_vals, packed):
    mesh = plsc.VectorSubcoreMesh(core_axis_name="c", subcore_axis_name="s")

    @functools.partial(
        pl.kernel,
        mesh=mesh,
        compiler_params=_sc_compiler_params(),
        out_type=jax.ShapeDtypeStruct((8, 128), jnp.float32),
        scratch_types=[
            pltpu.VMEM((EPW,), jnp.int32),
            pltpu.VMEM((EPW,), jnp.int32),
            pltpu.VMEM((EPW,), jnp.float32),
            [pltpu.VMEM((C, KP), jnp.int32) for _ in range(2 * DEPTH)],
            pltpu.VMEM((LANES,), jnp.float32),
            pltpu.VMEM_SHARED((N, KP), jnp.int32),
            [pltpu.SemaphoreType.DMA for _ in range(2 * DEPTH)],
        ],
    )
    def k(ei_hbm, a_hbm, packed_hbm, out_hbm,
          sidx, didx, av, rowbufs, accv, table, sems):
        sid = lax.axis_index("s")
        wid = sid * NC + lax.axis_index("c")
        base = wid * EPW
        row0 = sid * RPT

        pltpu.sync_copy(packed_hbm.at[pl.ds(row0, RPT)],
                        table.at[pl.ds(row0, RPT)])
        pltpu.sync_copy(ei_hbm.at[0, pl.ds(base, EPW)], sidx)
        pltpu.sync_copy(ei_hbm.at[1, pl.ds(base, EPW)], didx)
        pltpu.sync_copy(a_hbm.at[pl.ds(base, EPW)], av)
        accv[...] = jnp.zeros((LANES,), jnp.float32)
        plsc.subcore_barrier()

        bufs = tuple((rowbufs[2 * b], rowbufs[2 * b + 1],
                      sems[2 * b], sems[2 * b + 1]) for b in range(DEPTH))

        def issue(ci, sbuf, dbuf, ssem, dsem):
            pltpu.async_copy(table.at[sidx.at[pl.ds(ci * C, C)]], sbuf, ssem)
            pltpu.async_copy(table.at[didx.at[pl.ds(ci * C, C)]], dbuf, dsem)

        def wait(ci, sbuf, dbuf, ssem, dsem):
            pltpu.make_async_copy(
                table.at[sidx.at[pl.ds(ci * C, C)]], sbuf, ssem).wait()
            pltpu.make_async_copy(
                table.at[didx.at[pl.ds(ci * C, C)]], dbuf, dsem).wait()

        def compute(ci, sbuf, dbuf):
            def edge_group(g, acc):
                e0 = g * _UNROLL
                for u in range(_UNROLL):
                    e = e0 + u
                    t = jnp.zeros((LANES,), jnp.float32)
                    for j in range(KP // LANES):
                        sv = plsc.bitcast(
                            sbuf[e, pl.ds(j * LANES, LANES)], jnp.bfloat16)
                        dv = plsc.bitcast(
                            dbuf[e, pl.ds(j * LANES, LANES)], jnp.bfloat16)
                        p0, p1 = plsc.unpack(
                            sv * dv, format=plsc.PackFormat.INTERLEAVED)
                        t = t + p0 + p1
                    ab = plsc.load_gather(
                        av, [jnp.full((LANES,), ci * C + e, jnp.int32)])
                    acc = acc + ab * t
                return acc

            acc = lax.fori_loop(0, C // _UNROLL, edge_group,
                                jnp.zeros((LANES,), jnp.float32))
            accv[...] = accv[...] + acc

        for b in range(DEPTH):
            issue(b, *bufs[b])

        @pl.loop(0, NCHUNK, step=DEPTH)
        def _chunk(ci):
            for b in range(DEPTH):
                sbuf, dbuf, ssem, dsem = bufs[b]
                wait(ci + b, sbuf, dbuf, ssem, dsem)
                compute(ci + b, sbuf, dbuf)

                @pl.when(ci + b + DEPTH < NCHUNK)
                def _():
                    issue(ci + b + DEPTH, sbuf, dbuf, ssem, dsem)

        pltpu.sync_copy(accv,
                        out_hbm.at[wid % 8, pl.ds((wid // 8) * LANES, LANES)])

    return k(edge_index, a_vals, packed)


def _pack_body(e_ref, pk_ref):
    b = lax.bitcast_convert_type(e_ref[...], jnp.int32)
    r = _rne16(b)
    pk_ref[...] = jnp.bitwise_or(lax.shift_left(r[:, KP:], 16), r[:, :KP])


def _tc_pack(emb):
    return pl.pallas_call(
        _pack_body,
        grid=(N // _BR,),
        in_specs=[pl.BlockSpec((_BR, K), lambda i: (i, 0))],
        out_specs=pl.BlockSpec((_BR, KP), lambda i: (i, 0)),
        out_shape=jax.ShapeDtypeStruct((N, KP), jnp.int32),
    )(emb)


_BR = 1000


def _gram_body(e_ref, d_ref, ptp_ref, t2_ref):
    @pl.when(pl.program_id(0) == 0)
    def _():
        ptp_ref[...] = jnp.zeros((K, K), jnp.float32)
        t2_ref[0, 0] = 0.0

    blk = e_ref[...]
    ptp_ref[...] += lax.dot_general(blk, blk, (((0,), (0,)), ((), ())),
                                    preferred_element_type=jnp.float32)
    rs = jnp.sum(blk * blk, axis=1)
    drow = d_ref[pl.ds(pl.program_id(0), 1), :]
    t2_ref[0, 0] += jnp.sum(drow[0, :] * rs)


def _tc_gram(emb, d2):
    return pl.pallas_call(
        _gram_body,
        grid=(N // _BR,),
        in_specs=[
            pl.BlockSpec((_BR, K), lambda i: (i, 0)),
            pl.BlockSpec((N // _BR, _BR), lambda i: (0, 0)),
        ],
        out_specs=[
            pl.BlockSpec((K, K), lambda i: (0, 0)),
            pl.BlockSpec(memory_space=pltpu.SMEM),
        ],
        out_shape=[
            jax.ShapeDtypeStruct((K, K), jnp.float32),
            jax.ShapeDtypeStruct((1, 1), jnp.float32),
        ],
    )(emb, d2)


def _combine_body(ptp_ref, part_ref, t2_ref, out_ref):
    ptp = ptp_ref[...]
    term1 = jnp.sum(part_ref[...][:, :NW // 8 * LANES])
    term2 = t2_ref[0, 0]
    n = jnp.sqrt(jnp.sum(ptp * ptp))
    row = lax.broadcasted_iota(jnp.int32, (K, K), 0)
    col = lax.broadcasted_iota(jnp.int32, (K, K), 1)
    eye = jnp.where(row == col, jnp.float32(1.0), jnp.float32(0.0))
    m = ptp / n - eye / jnp.sqrt(jnp.float32(K))
    penalty = jnp.sqrt(jnp.sum(m * m))
    out_ref[0, 0] = -(term1 / term2) + penalty


def _tc_combine(ptp, partials, t2):
    return pl.pallas_call(
        _combine_body,
        in_specs=[
            pl.BlockSpec((K, K), lambda: (0, 0)),
            pl.BlockSpec((8, 128), lambda: (0, 0)),
            pl.BlockSpec(memory_space=pltpu.SMEM),
        ],
        out_specs=pl.BlockSpec(memory_space=pltpu.SMEM),
        out_shape=jax.ShapeDtypeStruct((1, 1), jnp.float32),
    )(ptp, partials, t2)


def kernel(embeddings, edge_index, edge_weight, normalized_A_values, D_values):
    del edge_weight
    packed = _tc_pack(embeddings)
    partials = _sc_term1(edge_index.astype(jnp.int32), normalized_A_values,
                         packed)
    ptp, t2 = _tc_gram(embeddings, D_values.reshape(N // _BR, _BR))
    out = _tc_combine(ptp, partials, t2)
    return out[0, 0]


# speedup vs baseline: 1.1003x; 1.0275x over previous
import dataclasses
import functools

import jax
import jax.numpy as jnp
from jax import lax
from jax.experimental import pallas as pl
from jax.experimental.pallas import tpu as pltpu
from jax.experimental.pallas import tpu_sc as plsc

N = 10000
E = 320000
K = 128

NC = 2
NS = 16
NW = NC * NS
EPW = E // NW
C = 40
NCHUNK = EPW // C
DEPTH = 5
LANES = 16
KP = K // 2
RPT = N // NS
ST = 25
NST = RPT // ST


def _sc_compiler_params():
    cp = pltpu.CompilerParams()
    if "needs_layout_passes" in pltpu.CompilerParams.__dataclass_fields__:
        cp = dataclasses.replace(cp, needs_layout_passes=False)
    if "use_tc_tiling_on_sc" in pltpu.CompilerParams.__dataclass_fields__:
        cp = dataclasses.replace(cp, use_tc_tiling_on_sc=False)
    return cp


_UNROLL = 5


def _rne16(b):
    odd = jnp.bitwise_and(lax.shift_right_logical(b, 16), 1)
    return lax.shift_right_logical(b + 0x7FFF + odd, 16)


def _sc_term1(edge_index, a_vals, packed):
    mesh = plsc.VectorSubcoreMesh(core_axis_name="c", subcore_axis_name="s")

    @functools.partial(
        pl.kernel,
        mesh=mesh,
        compiler_params=_sc_compiler_params(),
        out_type=jax.ShapeDtypeStruct((8, 128), jnp.float32),
        scratch_types=[
            pltpu.VMEM((EPW,), jnp.int32),
            pltpu.VMEM((EPW,), jnp.int32),
            pltpu.VMEM((EPW,), jnp.float32),
            [pltpu.VMEM((C, KP), jnp.int32) for _ in range(2 * DEPTH)],
            pltpu.VMEM((LANES,), jnp.float32),
            pltpu.VMEM_SHARED((N, KP), jnp.int32),
            [pltpu.SemaphoreType.DMA for _ in range(2 * DEPTH)],
        ],
    )
    def k(ei_hbm, a_hbm, packed_hbm, out_hbm,
          sidx, didx, av, rowbufs, accv, table, sems):
        sid = lax.axis_index("s")
        wid = sid * NC + lax.axis_index("c")
        base = wid * EPW
        row0 = sid * RPT

        pltpu.sync_copy(packed_hbm.at[pl.ds(row0, RPT), pl.ds(0, KP)],
                        table.at[pl.ds(row0, RPT)])
        pltpu.sync_copy(ei_hbm.at[0, pl.ds(base, EPW)], sidx)
        pltpu.sync_copy(ei_hbm.at[1, pl.ds(base, EPW)], didx)
        pltpu.sync_copy(a_hbm.at[pl.ds(base, EPW)], av)
        accv[...] = jnp.zeros((LANES,), jnp.float32)
        plsc.subcore_barrier()

        bufs = tuple((rowbufs[2 * b], rowbufs[2 * b + 1],
                      sems[2 * b], sems[2 * b + 1]) for b in range(DEPTH))

        def issue(ci, sbuf, dbuf, ssem, dsem):
            pltpu.async_copy(table.at[sidx.at[pl.ds(ci * C, C)]], sbuf, ssem)
            pltpu.async_copy(table.at[didx.at[pl.ds(ci * C, C)]], dbuf, dsem)

        def wait(ci, sbuf, dbuf, ssem, dsem):
            pltpu.make_async_copy(
                table.at[sidx.at[pl.ds(ci * C, C)]], sbuf, ssem).wait()
            pltpu.make_async_copy(
                table.at[didx.at[pl.ds(ci * C, C)]], dbuf, dsem).wait()

        def compute(ci, sbuf, dbuf):
            def edge_group(g, acc):
                e0 = g * _UNROLL
                for u in range(_UNROLL):
                    e = e0 + u
                    t = jnp.zeros((LANES,), jnp.float32)
                    for j in range(KP // LANES):
                        sv = plsc.bitcast(
                            sbuf[e, pl.ds(j * LANES, LANES)], jnp.bfloat16)
                        dv = plsc.bitcast(
                            dbuf[e, pl.ds(j * LANES, LANES)], jnp.bfloat16)
                        p0, p1 = plsc.unpack(
                            sv * dv, format=plsc.PackFormat.INTERLEAVED)
                        t = t + p0 + p1
                    ab = plsc.load_gather(
                        av, [jnp.full((LANES,), ci * C + e, jnp.int32)])
                    acc = acc + ab * t
                return acc

            acc = lax.fori_loop(0, C // _UNROLL, edge_group,
                                jnp.zeros((LANES,), jnp.float32))
            accv[...] = accv[...] + acc

        for b in range(DEPTH):
            issue(b, *bufs[b])

        @pl.loop(0, NCHUNK, step=DEPTH)
        def _chunk(ci):
            for b in range(DEPTH):
                sbuf, dbuf, ssem, dsem = bufs[b]
                wait(ci + b, sbuf, dbuf, ssem, dsem)
                compute(ci + b, sbuf, dbuf)

                @pl.when(ci + b + DEPTH < NCHUNK)
                def _():
                    issue(ci + b + DEPTH, sbuf, dbuf, ssem, dsem)

        pltpu.sync_copy(accv,
                        out_hbm.at[wid % 8, pl.ds((wid // 8) * LANES, LANES)])

    return k(edge_index, a_vals, packed)


def _pack_body(e_ref, pk_ref):
    b = lax.bitcast_convert_type(e_ref[...], jnp.int32)
    r = _rne16(b)
    pk_ref[:, :KP] = jnp.bitwise_or(lax.shift_left(r[:, KP:], 16), r[:, :KP])


def _tc_pack(emb):
    return pl.pallas_call(
        _pack_body,
        grid=(N // _BR,),
        in_specs=[pl.BlockSpec((_BR, K), lambda i: (i, 0))],
        out_specs=pl.BlockSpec((_BR, K), lambda i: (i, 0)),
        out_shape=jax.ShapeDtypeStruct((N, K), jnp.int32),
    )(emb)


_BR = 1000


def _gram_body(e_ref, d_ref, ptp_ref, t2_ref):
    @pl.when(pl.program_id(0) == 0)
    def _():
        ptp_ref[...] = jnp.zeros((K, K), jnp.float32)
        t2_ref[0, 0] = 0.0

    blk = e_ref[...]
    ptp_ref[...] += lax.dot_general(blk, blk, (((0,), (0,)), ((), ())),
                                    preferred_element_type=jnp.float32)
    rs = jnp.sum(blk * blk, axis=1)
    drow = d_ref[pl.ds(pl.program_id(0), 1), :]
    t2_ref[0, 0] += jnp.sum(drow[0, :] * rs)


def _tc_gram(emb, d2):
    return pl.pallas_call(
        _gram_body,
        grid=(N // _BR,),
        in_specs=[
            pl.BlockSpec((_BR, K), lambda i: (i, 0)),
            pl.BlockSpec((N // _BR, _BR), lambda i: (0, 0)),
        ],
        out_specs=[
            pl.BlockSpec((K, K), lambda i: (0, 0)),
            pl.BlockSpec(memory_space=pltpu.SMEM),
        ],
        out_shape=[
            jax.ShapeDtypeStruct((K, K), jnp.float32),
            jax.ShapeDtypeStruct((1, 1), jnp.float32),
        ],
    )(emb, d2)


def _combine_body(ptp_ref, part_ref, t2_ref, out_ref):
    ptp = ptp_ref[...]
    term1 = jnp.sum(part_ref[...][:, :NW // 8 * LANES])
    term2 = t2_ref[0, 0]
    n = jnp.sqrt(jnp.sum(ptp * ptp))
    row = lax.broadcasted_iota(jnp.int32, (K, K), 0)
    col = lax.broadcasted_iota(jnp.int32, (K, K), 1)
    eye = jnp.where(row == col, jnp.float32(1.0), jnp.float32(0.0))
    m = ptp / n - eye / jnp.sqrt(jnp.float32(K))
    penalty = jnp.sqrt(jnp.sum(m * m))
    out_ref[0, 0] = -(term1 / term2) + penalty


def _tc_combine(ptp, partials, t2):
    return pl.pallas_call(
        _combine_body,
        in_specs=[
            pl.BlockSpec((K, K), lambda: (0, 0)),
            pl.BlockSpec((8, 128), lambda: (0, 0)),
            pl.BlockSpec(memory_space=pltpu.SMEM),
        ],
        out_specs=pl.BlockSpec(memory_space=pltpu.SMEM),
        out_shape=jax.ShapeDtypeStruct((1, 1), jnp.float32),
    )(ptp, partials, t2)


def kernel(embeddings, edge_index, edge_weight, normalized_A_values, D_values):
    del edge_weight
    packed = _tc_pack(embeddings)
    partials = _sc_term1(edge_index.astype(jnp.int32), normalized_A_values,
                         packed)
    ptp, t2 = _tc_gram(embeddings, D_values.reshape(N // _BR, _BR))
    out = _tc_combine(ptp, partials, t2)
    return out[0, 0]
